# baseline (device time: 55463 ns/iter reference)
import jax
import jax.numpy as jnp
from jax import lax
from jax.experimental import pallas as pl
from jax.experimental.pallas import tpu as pltpu

N_DEV = 8
N_PEER = N_DEV - 1
T = 2
SCALE = 0.08838834764831843


def kernel(x, Wq, Wo, K_ext, V_ext):
    B, Sq, D = x.shape
    _, Skv, Hkv, Dh = K_ext.shape
    Hq = D // Dh
    G = Hkv
    HPG = Hq // Hkv
    U = T * G
    Sr = Sq // T
    R = HPG * Sr

    def body(x_ref, wq_ref, wo_ref, k_ref, v_ref, out_ref,
             send_o, recv_o, send_l, recv_l, kv_vmem,
             so_sem, ro_sem, sl_sem, rl_sem, kv_sem):
        my = lax.axis_index("i")
        peers = [my ^ d for d in range(1, N_DEV)]

        kv_dmas = []
        for g in range(G):
            for j, ref in enumerate((k_ref, v_ref)):
                dma = pltpu.make_async_copy(
                    ref.at[0, :, g, :], kv_vmem.at[j, g], kv_sem.at[j, g])
                dma.start()
                kv_dmas.append(dma)

        barrier = pltpu.get_barrier_semaphore()
        for p in peers:
            pl.semaphore_signal(barrier, inc=1, device_id=(p,),
                                device_id_type=pl.DeviceIdType.MESH)
        pl.semaphore_wait(barrier, N_PEER)

        def broadcast(u):
            rdmas = []
            for i, p in enumerate(peers):
                rdma_o = pltpu.make_async_remote_copy(
                    src_ref=send_o.at[u], dst_ref=recv_o.at[i, u],
                    send_sem=so_sem.at[i, u], recv_sem=ro_sem.at[i, u],
                    device_id=(p,), device_id_type=pl.DeviceIdType.MESH)
                rdma_l = pltpu.make_async_remote_copy(
                    src_ref=send_l.at[u], dst_ref=recv_l.at[i, u],
                    send_sem=sl_sem.at[i, u], recv_sem=rl_sem.at[i, u],
                    device_id=(p,), device_id_type=pl.DeviceIdType.MESH)
                rdma_o.start()
                rdma_l.start()
                rdmas.append((rdma_o, rdma_l))
            return rdmas

        q = jax.lax.dot_general(
            x_ref[...].astype(jnp.bfloat16), wq_ref[...].astype(jnp.bfloat16),
            (((1,), (0,)), ((), ())),
            preferred_element_type=jnp.float32) * SCALE
        qb = q.astype(jnp.bfloat16)

        for dma in kv_dmas:
            dma.wait()
        kb = [kv_vmem[0, g].astype(jnp.bfloat16) for g in range(G)]
        vb = [kv_vmem[1, g].astype(jnp.bfloat16) for g in range(G)]

        def local_partial(u):
            t, g = divmod(u, G)
            qu = jnp.concatenate(
                [qb[t * Sr:(t + 1) * Sr,
                    (g * HPG + j) * Dh:(g * HPG + j + 1) * Dh]
                 for j in range(HPG)], axis=0)
            s = jax.lax.dot_general(qu, kb[g], (((1,), (1,)), ((), ())),
                                    preferred_element_type=jnp.float32)
            p = jnp.exp(s.astype(jnp.bfloat16))
            l = jnp.sum(p, axis=1, dtype=jnp.float32)
            o = jax.lax.dot_general(p, vb[g], (((1,), (0,)), ((), ())),
                                    preferred_element_type=jnp.float32)
            send_o[u] = o.astype(jnp.bfloat16)
            send_l[u] = l
            return l, o

        L, O = [None] * U, [None] * U
        pending = [None] * U
        for u in range(U):
            L[u], O[u] = local_partial(u)
            pending[u] = broadcast(u)

        def merge(u):
            for rdma_o, rdma_l in pending[u]:
                rdma_o.wait()
                rdma_l.wait()
            L[u] = L[u] + jnp.sum(recv_l[:, u], axis=0)
            O[u] = O[u] + jnp.sum(recv_o[:, u].astype(jnp.float32), axis=0)

        def finish_half(t):
            blocks = []
            for h in range(Hq):
                g, j = divmod(h, HPG)
                u = t * G + g
                rows = slice(j * Sr, (j + 1) * Sr)
                blocks.append(O[u][rows] / L[u].reshape(R, 1)[rows])
            attn2d = jnp.concatenate(blocks, axis=1)
            out_ref[t * Sr:(t + 1) * Sr] = jax.lax.dot_general(
                attn2d.astype(jnp.bfloat16), wo_ref[...].astype(jnp.bfloat16),
                (((1,), (0,)), ((), ())), preferred_element_type=jnp.float32)

        for t in range(T):
            for g in range(G):
                merge(t * G + g)
            finish_half(t)

    out2d = pl.pallas_call(
        body,
        out_shape=jax.ShapeDtypeStruct((Sq, D), jnp.float32),
        in_specs=[pl.BlockSpec(memory_space=pltpu.VMEM)] * 3
        + [pl.BlockSpec(memory_space=pl.ANY)] * 2,
        out_specs=pl.BlockSpec(memory_space=pltpu.VMEM),
        scratch_shapes=[
            pltpu.VMEM((U, R, Dh), jnp.bfloat16),
            pltpu.VMEM((N_PEER, U, R, Dh), jnp.bfloat16),
            pltpu.VMEM((U, R), jnp.float32),
            pltpu.VMEM((N_PEER, U, R), jnp.float32),
            pltpu.VMEM((2, G, Skv, Dh), jnp.float32),
            pltpu.SemaphoreType.DMA((N_PEER, U)),
            pltpu.SemaphoreType.DMA((N_PEER, U)),
            pltpu.SemaphoreType.DMA((N_PEER, U)),
            pltpu.SemaphoreType.DMA((N_PEER, U)),
            pltpu.SemaphoreType.DMA((2, G)),
        ],
        compiler_params=pltpu.CompilerParams(collective_id=0),
    )(
        x.reshape(Sq, D),
        Wq,
        Wo,
        K_ext,
        V_ext,
    )
    return out2d.reshape(B, Sq, D)


# device time: 42281 ns/iter; 1.3118x vs baseline; 1.3118x over previous
import jax
import jax.numpy as jnp
from jax import lax
from jax.experimental import pallas as pl
from jax.experimental.pallas import tpu as pltpu

N_DEV = 8
N_PEER = N_DEV - 1
T = 4
SCALE = 0.08838834764831843


def kernel(x, Wq, Wo, K_ext, V_ext):
    B, Sq, D = x.shape
    _, Skv, Hkv, Dh = K_ext.shape
    Hq = D // Dh
    G = Hkv
    HPG = Hq // Hkv
    U = T * G
    Sr = Sq // T
    R = HPG * Sr

    def body(x_ref, wq_ref, wo_ref, k_ref, v_ref, out_ref,
             send_o, recv_o, send_l, recv_l,
             so_sem, ro_sem, sl_sem, rl_sem):
        my = lax.axis_index("i")
        peers = [my ^ d for d in range(1, N_DEV)]

        barrier = pltpu.get_barrier_semaphore()
        for p in peers:
            pl.semaphore_signal(barrier, inc=1, device_id=(p,),
                                device_id_type=pl.DeviceIdType.MESH)
        pl.semaphore_wait(barrier, N_PEER)

        def broadcast(u):
            rdmas = []
            for i, p in enumerate(peers):
                rdma_o = pltpu.make_async_remote_copy(
                    src_ref=send_o.at[u], dst_ref=recv_o.at[i, u],
                    send_sem=so_sem.at[i, u], recv_sem=ro_sem.at[i, u],
                    device_id=(p,), device_id_type=pl.DeviceIdType.MESH)
                rdma_l = pltpu.make_async_remote_copy(
                    src_ref=send_l.at[u], dst_ref=recv_l.at[i, u],
                    send_sem=sl_sem.at[i, u], recv_sem=rl_sem.at[i, u],
                    device_id=(p,), device_id_type=pl.DeviceIdType.MESH)
                rdma_o.start()
                rdma_l.start()
                rdmas.append((rdma_o, rdma_l))
            return rdmas

        q = jax.lax.dot_general(
            x_ref[0].astype(jnp.bfloat16), wq_ref[...].astype(jnp.bfloat16),
            (((1,), (0,)), ((), ())),
            preferred_element_type=jnp.float32) * SCALE
        qb = q.astype(jnp.bfloat16)

        kb = [k_ref[:, g * Dh:(g + 1) * Dh].astype(jnp.bfloat16)
              for g in range(G)]
        vb = [v_ref[:, g * Dh:(g + 1) * Dh].astype(jnp.bfloat16)
              for g in range(G)]

        def local_partial(u):
            t, g = divmod(u, G)
            qu = jnp.concatenate(
                [qb[t * Sr:(t + 1) * Sr,
                    (g * HPG + j) * Dh:(g * HPG + j + 1) * Dh]
                 for j in range(HPG)], axis=0)
            s = jax.lax.dot_general(qu, kb[g], (((1,), (1,)), ((), ())),
                                    preferred_element_type=jnp.float32)
            p = jnp.exp(s.astype(jnp.bfloat16))
            l = jnp.sum(p, axis=1, dtype=jnp.float32)
            o = jax.lax.dot_general(p, vb[g], (((1,), (0,)), ((), ())),
                                    preferred_element_type=jnp.float32)
            send_o[u] = o.astype(jnp.bfloat16)
            send_l[u] = l
            return l, o

        L, O = [None] * U, [None] * U
        pending = [None] * U
        for u in range(U):
            L[u], O[u] = local_partial(u)
            pending[u] = broadcast(u)

        def merge(u):
            for rdma_o, rdma_l in pending[u]:
                rdma_o.wait()
                rdma_l.wait()
            L[u] = L[u] + jnp.sum(recv_l[:, u], axis=0)
            O[u] = O[u] + jnp.sum(recv_o[:, u].astype(jnp.float32), axis=0)

        def finish_half(t):
            blocks = []
            for h in range(Hq):
                g, j = divmod(h, HPG)
                u = t * G + g
                rows = slice(j * Sr, (j + 1) * Sr)
                blocks.append(O[u][rows] / L[u].reshape(R, 1)[rows])
            attn2d = jnp.concatenate(blocks, axis=1)
            out_ref[0, t * Sr:(t + 1) * Sr] = jax.lax.dot_general(
                attn2d.astype(jnp.bfloat16), wo_ref[...].astype(jnp.bfloat16),
                (((1,), (0,)), ((), ())), preferred_element_type=jnp.float32)

        for t in range(T):
            for g in range(G):
                merge(t * G + g)
            finish_half(t)

    return pl.pallas_call(
        body,
        out_shape=jax.ShapeDtypeStruct((B, Sq, D), jnp.float32),
        in_specs=[pl.BlockSpec(memory_space=pltpu.VMEM)] * 5,
        out_specs=pl.BlockSpec(memory_space=pltpu.VMEM),
        scratch_shapes=[
            pltpu.VMEM((U, R, Dh), jnp.bfloat16),
            pltpu.VMEM((N_PEER, U, R, Dh), jnp.bfloat16),
            pltpu.VMEM((U, R), jnp.float32),
            pltpu.VMEM((N_PEER, U, R), jnp.float32),
            pltpu.SemaphoreType.DMA((N_PEER, U)),
            pltpu.SemaphoreType.DMA((N_PEER, U)),
            pltpu.SemaphoreType.DMA((N_PEER, U)),
            pltpu.SemaphoreType.DMA((N_PEER, U)),
        ],
        compiler_params=pltpu.CompilerParams(collective_id=0),
    )(
        x,
        Wq,
        Wo,
        K_ext.reshape(Skv, Hkv * Dh),
        V_ext.reshape(Skv, Hkv * Dh),
    )
